# baseline (speedup 1.0000x reference)
import jax
import jax.numpy as jnp
from jax.experimental import pallas as pl

N = 256
F_IN = 128
F_OUT = 256
BI = 8


def _fused_body(x1_ref, x2_ref, w_ref, b_ref, out_ref):
    w = w_ref[...]
    bb = b_ref[0]
    for k in range(BI):
        a1 = x1_ref[k]
        a2 = x2_ref[:, k, :]
        e1 = jnp.dot(a1, w, preferred_element_type=jnp.float32) + bb
        e2 = jnp.dot(a2, w, preferred_element_type=jnp.float32) + bb
        out_ref[k] = jnp.dot(e1, e2, preferred_element_type=jnp.float32)


def kernel(x, W, b):
    x1 = x[0, 0]
    x2 = x[0, 1]
    b2 = b.reshape(1, F_OUT)
    out = pl.pallas_call(
        _fused_body,
        grid=(N // BI,),
        in_specs=[
            pl.BlockSpec((BI, N, F_IN), lambda i: (i, 0, 0)),
            pl.BlockSpec((N, BI, F_IN), lambda i: (0, i, 0)),
            pl.BlockSpec((F_IN, F_OUT), lambda i: (0, 0)),
            pl.BlockSpec((1, F_OUT), lambda i: (0, 0)),
        ],
        out_specs=pl.BlockSpec((BI, N, F_OUT), lambda i: (i, 0, 0)),
        out_shape=jax.ShapeDtypeStruct((N, N, F_OUT), jnp.float32),
    )(x1, x2, W, b2)
    return out[None]


# speedup vs baseline: 1.2029x; 1.2029x over previous
import jax
import jax.numpy as jnp
from jax.experimental import pallas as pl

N = 256
F_IN = 128
F_OUT = 256
BI = 8


def _fused_body(x1_ref, x2_ref, w_ref, b_ref, out_ref):
    w = w_ref[...]
    bb = b_ref[0]
    e1b = jax.lax.dot_general(
        x1_ref[...], w, (((2,), (0,)), ((), ())),
        preferred_element_type=jnp.float32) + bb
    e2b = jax.lax.dot_general(
        x2_ref[...], w, (((2,), (0,)), ((), ())),
        preferred_element_type=jnp.float32) + bb
    out_ref[...] = jax.lax.dot_general(
        e1b, e2b, (((2,), (0,)), ((0,), (1,))),
        preferred_element_type=jnp.float32)


def kernel(x, W, b):
    x1 = x[0, 0]
    x2 = x[0, 1]
    b2 = b.reshape(1, F_OUT)
    out = pl.pallas_call(
        _fused_body,
        grid=(N // BI,),
        in_specs=[
            pl.BlockSpec((BI, N, F_IN), lambda i: (i, 0, 0)),
            pl.BlockSpec((N, BI, F_IN), lambda i: (0, i, 0)),
            pl.BlockSpec((F_IN, F_OUT), lambda i: (0, 0)),
            pl.BlockSpec((1, F_OUT), lambda i: (0, 0)),
        ],
        out_specs=pl.BlockSpec((BI, N, F_OUT), lambda i: (i, 0, 0)),
        out_shape=jax.ShapeDtypeStruct((N, N, F_OUT), jnp.float32),
    )(x1, x2, W, b2)
    return out[None]
